# baseline (speedup 1.0000x reference)
import functools

import jax
import jax.numpy as jnp
from jax import lax
from jax.experimental import pallas as pl
from jax.experimental.pallas import tpu as pltpu
from jax.experimental.pallas import tpu_sc as plsc

B = 16384
DIM = 64
C = 1000
D = 8
MARGIN = 0.1

TBL = 8192
NC = 2
NS = 16
NW = NC * NS
PER_TILE = B // NW
ROWS = PER_TILE // 128
SLICE = TBL // NS
ZR = 128


def _segment_kernel(emb_hbm, comp_hbm, dose_hbm, sums_out, counts_out,
                    emb_v, cl_v, dl_v, idx_v, ones_v, zrow_v, zblk_v,
                    sh_sums, sh_counts):
    c = lax.axis_index("c")
    s = lax.axis_index("s")
    wid = c * NS + s
    base = wid * PER_TILE

    for r in range(ZR):
        for l in range(DIM // 16):
            zblk_v[r, pl.ds(l * 16, 16)] = jnp.zeros((16,), jnp.float32)
    def _zrow(i, _):
        zrow_v[pl.ds(i * 16, 16)] = jnp.zeros((16,), jnp.float32)
        return 0
    lax.fori_loop(0, SLICE // 16, _zrow, 0)
    for q in range(SLICE // ZR):
        pltpu.sync_copy(zblk_v, sh_sums.at[pl.ds(s * SLICE + q * ZR, ZR)])
    pltpu.sync_copy(zrow_v, sh_counts.at[pl.ds(s * SLICE, SLICE)])

    for i in range(128 // 16):
        ones_v[pl.ds(i * 16, 16)] = jnp.ones((16,), jnp.float32)

    pltpu.sync_copy(emb_hbm.at[pl.ds(base, PER_TILE)], emb_v)
    pltpu.sync_copy(comp_hbm.at[pl.ds(base, PER_TILE)], cl_v)
    pltpu.sync_copy(dose_hbm.at[pl.ds(base, PER_TILE)], dl_v)
    for j in range(ROWS):
        for l in range(128 // 16):
            o = (j * 128 + l * 16)
            idx_v[j, pl.ds(l * 16, 16)] = (dl_v[pl.ds(o, 16)] * 1024
                                           + cl_v[pl.ds(o, 16)])

    plsc.subcore_barrier()

    for j in range(ROWS):
        pltpu.sync_copy(emb_v.at[pl.ds(j * 128, 128)],
                        sh_sums.at[idx_v.at[j]], add=True)
        pltpu.sync_copy(ones_v, sh_counts.at[idx_v.at[j]], add=True)

    plsc.subcore_barrier()

    pltpu.sync_copy(sh_sums.at[pl.ds(s * SLICE, SLICE)], emb_v)
    pltpu.sync_copy(emb_v, sums_out.at[c, pl.ds(s * SLICE, SLICE)])
    pltpu.sync_copy(sh_counts.at[pl.ds(s * SLICE, SLICE)], zrow_v)
    pltpu.sync_copy(zrow_v, counts_out.at[c, pl.ds(s * SLICE, SLICE)])


def _loss_kernel(sums_ref, counts_ref, out_ref):
    E = sums_ref[0] + sums_ref[1]
    cnt = counts_ref[0] + counts_ref[1]
    s64 = jnp.sum(jnp.sum(E, axis=0), axis=0)
    mean = s64 * (1.0 / B)
    norm = jnp.sqrt(jnp.sum(mean * mean))
    u = mean / jnp.maximum(norm, 1e-12)
    dots = jnp.sum(E * u[None, None, :], axis=2)
    present = cnt > 0.0
    means = jnp.where(present, 1.0 - dots / jnp.maximum(cnt, 1.0), 0.0)
    ploss = jnp.zeros((1, 1024), jnp.float32)
    pcnt = jnp.zeros((1, 1024), jnp.float32)
    for dl in range(D - 1):
        for dh in range(dl + 1, D):
            valid = present[dl:dl + 1] & present[dh:dh + 1]
            for m in range(dl + 1, dh):
                valid = valid & jnp.logical_not(present[m:m + 1])
            viol = MARGIN - (means[dh:dh + 1] - means[dl:dl + 1])
            ploss = ploss + jnp.where(valid, jnp.maximum(viol, 0.0), 0.0)
            pcnt = pcnt + valid.astype(jnp.float32)
    loss = jnp.sum(ploss)
    cnt_t = jnp.sum(pcnt)
    out_ref[...] = jnp.where(cnt_t > 0.0, loss / jnp.maximum(cnt_t, 1.0),
                             0.0)[None, None]


def kernel(embeddings, compound_labels, dose_labels):
    comp = compound_labels.astype(jnp.int32)
    dose = dose_labels.astype(jnp.int32)

    seg = functools.partial(
        pl.kernel,
        mesh=plsc.VectorSubcoreMesh(core_axis_name="c", subcore_axis_name="s"),
        compiler_params=pltpu.CompilerParams(use_tc_tiling_on_sc=False),
        out_type=(
            jax.ShapeDtypeStruct((NC, TBL, DIM), jnp.float32),
            jax.ShapeDtypeStruct((NC, TBL), jnp.float32),
        ),
        scratch_types=[
            pltpu.VMEM((PER_TILE, DIM), jnp.float32),
            pltpu.VMEM((PER_TILE,), jnp.int32),
            pltpu.VMEM((PER_TILE,), jnp.int32),
            pltpu.VMEM((ROWS, 128), jnp.int32),
            pltpu.VMEM((128,), jnp.float32),
            pltpu.VMEM((SLICE,), jnp.float32),
            pltpu.VMEM((ZR, DIM), jnp.float32),
            pltpu.VMEM_SHARED((TBL, DIM), jnp.float32),
            pltpu.VMEM_SHARED((TBL,), jnp.float32),
        ],
    )(_segment_kernel)
    sums4, counts3 = seg(embeddings, comp, dose)

    out = pl.pallas_call(
        _loss_kernel,
        out_shape=jax.ShapeDtypeStruct((1, 1), jnp.float32),
    )(sums4.reshape(NC, D, 1024, DIM), counts3.reshape(NC, D, 1024))
    return out.reshape(())


# speedup vs baseline: 1.5294x; 1.5294x over previous
import functools

import jax
import jax.numpy as jnp
from jax import lax
from jax.experimental import pallas as pl
from jax.experimental.pallas import tpu as pltpu
from jax.experimental.pallas import tpu_sc as plsc

B = 16384
DIM = 64
C = 1000
D = 8
MARGIN = 0.1

TBL = 8192
NC = 2
NS = 16
NW = NC * NS
PER_TILE = B // NW
ROWS = PER_TILE // 128
SLICE = TBL // NS


def _dist_kernel(emb_ref, dist_ref):
    emb = emb_ref[...].reshape(128, 128, DIM)
    col = jnp.sum(emb, axis=1)
    s = jnp.sum(col, axis=0)
    mean = s * (1.0 / B)
    norm = jnp.sqrt(jnp.sum(mean * mean))
    u = mean / jnp.maximum(norm, 1e-12)
    dist_ref[...] = 1.0 - jnp.sum(emb * u[None, None, :], axis=2)


def _segment_kernel(dist_hbm, comp_hbm, dose_hbm, sums_out, counts_out,
                    val_v, cl_v, dl_v, idx_v, ones_v, zrow_v,
                    sh_sums, sh_counts):
    c = lax.axis_index("c")
    s = lax.axis_index("s")
    wid = c * NS + s
    base = wid * PER_TILE

    def _zrow(i, _):
        zrow_v[pl.ds(i * 16, 16)] = jnp.zeros((16,), jnp.float32)
        return 0
    lax.fori_loop(0, SLICE // 16, _zrow, 0)
    pltpu.sync_copy(zrow_v, sh_sums.at[pl.ds(s * SLICE, SLICE)])
    pltpu.sync_copy(zrow_v, sh_counts.at[pl.ds(s * SLICE, SLICE)])

    for i in range(128 // 16):
        ones_v[pl.ds(i * 16, 16)] = jnp.ones((16,), jnp.float32)

    pltpu.sync_copy(dist_hbm.at[pl.ds(base, PER_TILE)], val_v)
    pltpu.sync_copy(comp_hbm.at[pl.ds(base, PER_TILE)], cl_v)
    pltpu.sync_copy(dose_hbm.at[pl.ds(base, PER_TILE)], dl_v)
    for j in range(ROWS):
        for l in range(128 // 16):
            o = (j * 128 + l * 16)
            idx_v[j, pl.ds(l * 16, 16)] = (dl_v[pl.ds(o, 16)] * 1024
                                           + cl_v[pl.ds(o, 16)])

    plsc.subcore_barrier()

    for j in range(ROWS):
        pltpu.sync_copy(val_v.at[pl.ds(j * 128, 128)],
                        sh_sums.at[idx_v.at[j]], add=True)
        pltpu.sync_copy(ones_v, sh_counts.at[idx_v.at[j]], add=True)

    plsc.subcore_barrier()

    pltpu.sync_copy(sh_sums.at[pl.ds(s * SLICE, SLICE)], zrow_v)
    pltpu.sync_copy(zrow_v, sums_out.at[c, pl.ds(s * SLICE, SLICE)])
    pltpu.sync_copy(sh_counts.at[pl.ds(s * SLICE, SLICE)], zrow_v)
    pltpu.sync_copy(zrow_v, counts_out.at[c, pl.ds(s * SLICE, SLICE)])


def _loss_kernel(sums_ref, counts_ref, out_ref):
    sums = sums_ref[0] + sums_ref[1]
    counts = counts_ref[0] + counts_ref[1]
    present = counts > 0.0
    means = jnp.where(present, sums / jnp.maximum(counts, 1.0), 0.0)
    ploss = jnp.zeros((1, 1024), jnp.float32)
    pcnt = jnp.zeros((1, 1024), jnp.float32)
    for dl in range(D - 1):
        for dh in range(dl + 1, D):
            valid = present[dl:dl + 1] & present[dh:dh + 1]
            for m in range(dl + 1, dh):
                valid = valid & jnp.logical_not(present[m:m + 1])
            viol = MARGIN - (means[dh:dh + 1] - means[dl:dl + 1])
            ploss = ploss + jnp.where(valid, jnp.maximum(viol, 0.0), 0.0)
            pcnt = pcnt + valid.astype(jnp.float32)
    loss = jnp.sum(ploss)
    cnt = jnp.sum(pcnt)
    out_ref[...] = jnp.where(cnt > 0.0, loss / jnp.maximum(cnt, 1.0),
                             0.0)[None, None]


def kernel(embeddings, compound_labels, dose_labels):
    comp = compound_labels.astype(jnp.int32)
    dose = dose_labels.astype(jnp.int32)

    dist = pl.pallas_call(
        _dist_kernel,
        out_shape=jax.ShapeDtypeStruct((128, 128), jnp.float32),
    )(embeddings)

    seg = functools.partial(
        pl.kernel,
        mesh=plsc.VectorSubcoreMesh(core_axis_name="c", subcore_axis_name="s"),
        compiler_params=pltpu.CompilerParams(use_tc_tiling_on_sc=False),
        out_type=(
            jax.ShapeDtypeStruct((NC, TBL), jnp.float32),
            jax.ShapeDtypeStruct((NC, TBL), jnp.float32),
        ),
        scratch_types=[
            pltpu.VMEM((PER_TILE,), jnp.float32),
            pltpu.VMEM((PER_TILE,), jnp.int32),
            pltpu.VMEM((PER_TILE,), jnp.int32),
            pltpu.VMEM((ROWS, 128), jnp.int32),
            pltpu.VMEM((128,), jnp.float32),
            pltpu.VMEM((SLICE,), jnp.float32),
            pltpu.VMEM_SHARED((TBL,), jnp.float32),
            pltpu.VMEM_SHARED((TBL,), jnp.float32),
        ],
    )(_segment_kernel)
    sums2, counts2 = seg(dist.reshape(B), comp, dose)

    out = pl.pallas_call(
        _loss_kernel,
        out_shape=jax.ShapeDtypeStruct((1, 1), jnp.float32),
    )(sums2.reshape(NC, D, 1024), counts2.reshape(NC, D, 1024))
    return out.reshape(())


# speedup vs baseline: 2.0457x; 1.3376x over previous
import functools

import jax
import jax.numpy as jnp
from jax import lax
from jax.experimental import pallas as pl
from jax.experimental.pallas import tpu as pltpu
from jax.experimental.pallas import tpu_sc as plsc

B = 16384
DIM = 64
C = 1000
D = 8
MARGIN = 0.1

TBL = 8192
NC = 2
NS = 16
NW = NC * NS
PER_TILE = B // NW
ROWS = PER_TILE // 128
SLICE = TBL // NS


def _dist_kernel(embt_ref, dist_ref):
    et = embt_ref[...]
    s = jnp.sum(et, axis=1)
    mean = s * (1.0 / B)
    norm = jnp.sqrt(jnp.sum(mean * mean))
    u = mean / jnp.maximum(norm, 1e-12)
    v3 = et.reshape(DIM, 128, 128)
    dist_ref[...] = 1.0 - jnp.sum(v3 * u[:, None, None], axis=0)


def _segment_kernel(dist_hbm, comp_hbm, dose_hbm, sums_out, counts_out,
                    val_v, cl_v, dl_v, idx_v, ones_v, zrow_v,
                    sh_sums, sh_counts):
    c = lax.axis_index("c")
    s = lax.axis_index("s")
    wid = c * NS + s
    base = wid * PER_TILE

    def _zrow(i, _):
        zrow_v[pl.ds(i * 16, 16)] = jnp.zeros((16,), jnp.float32)
        return 0
    lax.fori_loop(0, SLICE // 16, _zrow, 0)
    pltpu.sync_copy(zrow_v, sh_sums.at[pl.ds(s * SLICE, SLICE)])
    pltpu.sync_copy(zrow_v, sh_counts.at[pl.ds(s * SLICE, SLICE)])

    for i in range(128 // 16):
        ones_v[pl.ds(i * 16, 16)] = jnp.ones((16,), jnp.float32)

    pltpu.sync_copy(dist_hbm.at[pl.ds(base, PER_TILE)], val_v)
    pltpu.sync_copy(comp_hbm.at[pl.ds(base, PER_TILE)], cl_v)
    pltpu.sync_copy(dose_hbm.at[pl.ds(base, PER_TILE)], dl_v)
    for j in range(ROWS):
        for l in range(128 // 16):
            o = (j * 128 + l * 16)
            idx_v[j, pl.ds(l * 16, 16)] = (dl_v[pl.ds(o, 16)] * 1024
                                           + cl_v[pl.ds(o, 16)])

    plsc.subcore_barrier()

    for j in range(ROWS):
        pltpu.sync_copy(val_v.at[pl.ds(j * 128, 128)],
                        sh_sums.at[idx_v.at[j]], add=True)
        pltpu.sync_copy(ones_v, sh_counts.at[idx_v.at[j]], add=True)

    plsc.subcore_barrier()

    pltpu.sync_copy(sh_sums.at[pl.ds(s * SLICE, SLICE)], zrow_v)
    pltpu.sync_copy(zrow_v, sums_out.at[c, pl.ds(s * SLICE, SLICE)])
    pltpu.sync_copy(sh_counts.at[pl.ds(s * SLICE, SLICE)], zrow_v)
    pltpu.sync_copy(zrow_v, counts_out.at[c, pl.ds(s * SLICE, SLICE)])


def _loss_kernel(sums_ref, counts_ref, out_ref):
    sums = sums_ref[0:1] + sums_ref[1:2]
    counts = counts_ref[0:1] + counts_ref[1:2]
    present = [counts[:, d * 1024:(d + 1) * 1024] > 0.0 for d in range(D)]
    means = [jnp.where(present[d],
                       sums[:, d * 1024:(d + 1) * 1024]
                       / jnp.maximum(counts[:, d * 1024:(d + 1) * 1024], 1.0),
                       0.0) for d in range(D)]
    ploss = jnp.zeros((1, 1024), jnp.float32)
    pcnt = jnp.zeros((1, 1024), jnp.float32)
    for dl in range(D - 1):
        for dh in range(dl + 1, D):
            valid = present[dl] & present[dh]
            for m in range(dl + 1, dh):
                valid = valid & jnp.logical_not(present[m])
            viol = MARGIN - (means[dh] - means[dl])
            ploss = ploss + jnp.where(valid, jnp.maximum(viol, 0.0), 0.0)
            pcnt = pcnt + valid.astype(jnp.float32)
    loss = jnp.sum(ploss)
    cnt = jnp.sum(pcnt)
    out_ref[...] = jnp.where(cnt > 0.0, loss / jnp.maximum(cnt, 1.0),
                             0.0)[None, None]


def kernel(embeddings, compound_labels, dose_labels):
    comp = compound_labels.astype(jnp.int32)
    dose = dose_labels.astype(jnp.int32)

    dist = pl.pallas_call(
        _dist_kernel,
        out_shape=jax.ShapeDtypeStruct((128, 128), jnp.float32),
    )(embeddings.T)

    seg = functools.partial(
        pl.kernel,
        mesh=plsc.VectorSubcoreMesh(core_axis_name="c", subcore_axis_name="s"),
        compiler_params=pltpu.CompilerParams(use_tc_tiling_on_sc=False),
        out_type=(
            jax.ShapeDtypeStruct((NC, TBL), jnp.float32),
            jax.ShapeDtypeStruct((NC, TBL), jnp.float32),
        ),
        scratch_types=[
            pltpu.VMEM((PER_TILE,), jnp.float32),
            pltpu.VMEM((PER_TILE,), jnp.int32),
            pltpu.VMEM((PER_TILE,), jnp.int32),
            pltpu.VMEM((ROWS, 128), jnp.int32),
            pltpu.VMEM((128,), jnp.float32),
            pltpu.VMEM((SLICE,), jnp.float32),
            pltpu.VMEM_SHARED((TBL,), jnp.float32),
            pltpu.VMEM_SHARED((TBL,), jnp.float32),
        ],
    )(_segment_kernel)
    sums2, counts2 = seg(dist.reshape(B), comp, dose)

    out = pl.pallas_call(
        _loss_kernel,
        out_shape=jax.ShapeDtypeStruct((1, 1), jnp.float32),
    )(sums2, counts2)
    return out.reshape(())


# speedup vs baseline: 2.2850x; 1.1170x over previous
import functools

import jax
import jax.numpy as jnp
from jax import lax
from jax.experimental import pallas as pl
from jax.experimental.pallas import tpu as pltpu
from jax.experimental.pallas import tpu_sc as plsc

B = 16384
DIM = 64
C = 1000
D = 8
MARGIN = 0.1

TBL = 8192
NC = 2
NS = 16
NW = NC * NS
PER_TILE = B // NW
ROWS = PER_TILE // 128
SLICE = TBL // NS


def _dist_kernel(embt_ref, dist_ref):
    et = embt_ref[...]
    s = jnp.sum(et, axis=1)
    mean = s * (1.0 / B)
    norm = jnp.sqrt(jnp.sum(mean * mean))
    u = mean / jnp.maximum(norm, 1e-12)
    v3 = et.reshape(DIM, 128, 128)
    dist_ref[...] = 1.0 - jnp.sum(v3 * u[:, None, None], axis=0)


def _segment_kernel(dist_hbm, comp_hbm, dose_hbm, sums_out, counts_out,
                    val_v, cl_v, dl_v, idx_v, ones_v, zrow_v,
                    sh_sums, sh_counts):
    c = lax.axis_index("c")
    s = lax.axis_index("s")
    wid = c * NS + s
    base = wid * PER_TILE

    def _zrow(i, _):
        zrow_v[pl.ds(i * 16, 16)] = jnp.zeros((16,), jnp.float32)
        return 0
    lax.fori_loop(0, SLICE // 16, _zrow, 0)
    pltpu.sync_copy(zrow_v, sh_sums.at[pl.ds(s * SLICE, SLICE)])
    pltpu.sync_copy(zrow_v, sh_counts.at[pl.ds(s * SLICE, SLICE)])

    for i in range(128 // 16):
        ones_v[pl.ds(i * 16, 16)] = jnp.ones((16,), jnp.float32)

    pltpu.sync_copy(dist_hbm.at[pl.ds(base, PER_TILE)], val_v)
    pltpu.sync_copy(comp_hbm.at[pl.ds(base, PER_TILE)], cl_v)
    pltpu.sync_copy(dose_hbm.at[pl.ds(base, PER_TILE)], dl_v)
    for j in range(ROWS):
        for l in range(128 // 16):
            o = (j * 128 + l * 16)
            idx_v[j, pl.ds(l * 16, 16)] = (dl_v[pl.ds(o, 16)] * 1024
                                           + cl_v[pl.ds(o, 16)])

    plsc.subcore_barrier()

    for j in range(ROWS):
        pltpu.sync_copy(val_v.at[pl.ds(j * 128, 128)],
                        sh_sums.at[idx_v.at[j]], add=True)
        pltpu.sync_copy(ones_v, sh_counts.at[idx_v.at[j]], add=True)

    plsc.subcore_barrier()

    off = c * TBL + s * SLICE
    pltpu.sync_copy(sh_sums.at[pl.ds(s * SLICE, SLICE)], zrow_v)
    pltpu.sync_copy(zrow_v, sums_out.at[pl.ds(off, SLICE)])
    pltpu.sync_copy(sh_counts.at[pl.ds(s * SLICE, SLICE)], zrow_v)
    pltpu.sync_copy(zrow_v, counts_out.at[pl.ds(off, SLICE)])


def _loss_kernel(sums_ref, counts_ref, out_ref):
    su = sums_ref[...]
    co = counts_ref[...]
    present = []
    means = []
    for d in range(D):
        s_d = su[d * 8:(d + 1) * 8, :] + su[64 + d * 8:64 + (d + 1) * 8, :]
        c_d = co[d * 8:(d + 1) * 8, :] + co[64 + d * 8:64 + (d + 1) * 8, :]
        p_d = c_d > 0.0
        present.append(p_d)
        means.append(jnp.where(p_d, s_d / jnp.maximum(c_d, 1.0), 0.0))
    ploss = jnp.zeros((8, 128), jnp.float32)
    pcnt = jnp.zeros((8, 128), jnp.float32)
    for dl in range(D - 1):
        for dh in range(dl + 1, D):
            valid = present[dl] & present[dh]
            for m in range(dl + 1, dh):
                valid = valid & jnp.logical_not(present[m])
            viol = MARGIN - (means[dh] - means[dl])
            ploss = ploss + jnp.where(valid, jnp.maximum(viol, 0.0), 0.0)
            pcnt = pcnt + valid.astype(jnp.float32)
    loss = jnp.sum(ploss)
    cnt = jnp.sum(pcnt)
    out_ref[...] = jnp.where(cnt > 0.0, loss / jnp.maximum(cnt, 1.0),
                             0.0)[None, None]


def kernel(embeddings, compound_labels, dose_labels):
    comp = compound_labels.astype(jnp.int32)
    dose = dose_labels.astype(jnp.int32)

    dist = pl.pallas_call(
        _dist_kernel,
        out_shape=jax.ShapeDtypeStruct((128, 128), jnp.float32),
    )(embeddings.T)

    seg = functools.partial(
        pl.kernel,
        mesh=plsc.VectorSubcoreMesh(core_axis_name="c", subcore_axis_name="s"),
        compiler_params=pltpu.CompilerParams(use_tc_tiling_on_sc=False),
        out_type=(
            jax.ShapeDtypeStruct((NC * TBL,), jnp.float32),
            jax.ShapeDtypeStruct((NC * TBL,), jnp.float32),
        ),
        scratch_types=[
            pltpu.VMEM((PER_TILE,), jnp.float32),
            pltpu.VMEM((PER_TILE,), jnp.int32),
            pltpu.VMEM((PER_TILE,), jnp.int32),
            pltpu.VMEM((ROWS, 128), jnp.int32),
            pltpu.VMEM((128,), jnp.float32),
            pltpu.VMEM((SLICE,), jnp.float32),
            pltpu.VMEM_SHARED((TBL,), jnp.float32),
            pltpu.VMEM_SHARED((TBL,), jnp.float32),
        ],
    )(_segment_kernel)
    sums2, counts2 = seg(dist.reshape(B), comp, dose)

    out = pl.pallas_call(
        _loss_kernel,
        out_shape=jax.ShapeDtypeStruct((1, 1), jnp.float32),
    )(sums2.reshape(128, 128), counts2.reshape(128, 128))
    return out.reshape(())


# speedup vs baseline: 2.3292x; 1.0194x over previous
import functools

import jax
import jax.numpy as jnp
from jax import lax
from jax.experimental import pallas as pl
from jax.experimental.pallas import tpu as pltpu
from jax.experimental.pallas import tpu_sc as plsc

B = 16384
DIM = 64
C = 1000
D = 8
MARGIN = 0.1

TBL = 8192
NC = 2
NS = 16
NW = NC * NS
PER_TILE = B // NW
ROWS = PER_TILE // 128
SLICE = TBL // NS


def _dist_kernel(embt_ref, comp_ref, dose_ref, dist_ref, key_ref):
    et = embt_ref[...]
    s = jnp.sum(et, axis=1)
    mean = s * (1.0 / B)
    norm = jnp.sqrt(jnp.sum(mean * mean))
    u = mean / jnp.maximum(norm, 1e-12)
    v3 = et.reshape(DIM, 128, 128)
    dist_ref[...] = 1.0 - jnp.sum(v3 * u[:, None, None], axis=0)
    key_ref[...] = dose_ref[...] * 1024 + comp_ref[...]


def _segment_kernel(dist_hbm, key_hbm, sums_out, counts_out,
                    val_v, idx_v, ones_v, zrow_v, zrow2_v,
                    sh_sums, sh_counts):
    c = lax.axis_index("c")
    s = lax.axis_index("s")
    wid = c * NS + s

    def _zrow(i, _):
        zrow_v[pl.ds(i * 16, 16)] = jnp.zeros((16,), jnp.float32)
        return 0
    lax.fori_loop(0, SLICE // 16, _zrow, 0)
    for i in range(128 // 16):
        ones_v[pl.ds(i * 16, 16)] = jnp.ones((16,), jnp.float32)
    pltpu.sync_copy(zrow_v, sh_sums.at[pl.ds(s * SLICE, SLICE)])
    pltpu.sync_copy(zrow_v, sh_counts.at[pl.ds(s * SLICE, SLICE)])
    pltpu.sync_copy(dist_hbm.at[wid], val_v)
    pltpu.sync_copy(key_hbm.at[wid], idx_v)

    plsc.subcore_barrier()

    for j in range(ROWS):
        pltpu.sync_copy(val_v.at[j], sh_sums.at[idx_v.at[j]], add=True)
        pltpu.sync_copy(ones_v, sh_counts.at[idx_v.at[j]], add=True)

    plsc.subcore_barrier()

    off = c * TBL + s * SLICE
    pltpu.sync_copy(sh_sums.at[pl.ds(s * SLICE, SLICE)], zrow_v)
    pltpu.sync_copy(zrow_v, sums_out.at[pl.ds(off, SLICE)])
    pltpu.sync_copy(sh_counts.at[pl.ds(s * SLICE, SLICE)], zrow2_v)
    pltpu.sync_copy(zrow2_v, counts_out.at[pl.ds(off, SLICE)])


def _loss_kernel(sums_ref, counts_ref, out_ref):
    su = sums_ref[...]
    co = counts_ref[...]
    present = []
    means = []
    for d in range(D):
        s_d = su[d * 8:(d + 1) * 8, :] + su[64 + d * 8:64 + (d + 1) * 8, :]
        c_d = co[d * 8:(d + 1) * 8, :] + co[64 + d * 8:64 + (d + 1) * 8, :]
        p_d = c_d > 0.0
        present.append(p_d)
        means.append(jnp.where(p_d, s_d / jnp.maximum(c_d, 1.0), 0.0))
    ploss = jnp.zeros((8, 128), jnp.float32)
    pcnt = jnp.zeros((8, 128), jnp.float32)
    for dl in range(D - 1):
        for dh in range(dl + 1, D):
            valid = present[dl] & present[dh]
            for m in range(dl + 1, dh):
                valid = valid & jnp.logical_not(present[m])
            viol = MARGIN - (means[dh] - means[dl])
            ploss = ploss + jnp.where(valid, jnp.maximum(viol, 0.0), 0.0)
            pcnt = pcnt + valid.astype(jnp.float32)
    loss = jnp.sum(ploss)
    cnt = jnp.sum(pcnt)
    out_ref[...] = jnp.where(cnt > 0.0, loss / jnp.maximum(cnt, 1.0),
                             0.0)[None, None]


def kernel(embeddings, compound_labels, dose_labels):
    comp = compound_labels.astype(jnp.int32).reshape(128, 128)
    dose = dose_labels.astype(jnp.int32).reshape(128, 128)

    dist, keys = pl.pallas_call(
        _dist_kernel,
        out_shape=(
            jax.ShapeDtypeStruct((128, 128), jnp.float32),
            jax.ShapeDtypeStruct((128, 128), jnp.int32),
        ),
    )(embeddings.T, comp, dose)

    seg = functools.partial(
        pl.kernel,
        mesh=plsc.VectorSubcoreMesh(core_axis_name="c", subcore_axis_name="s"),
        compiler_params=pltpu.CompilerParams(use_tc_tiling_on_sc=False),
        out_type=(
            jax.ShapeDtypeStruct((NC * TBL,), jnp.float32),
            jax.ShapeDtypeStruct((NC * TBL,), jnp.float32),
        ),
        scratch_types=[
            pltpu.VMEM((ROWS, 128), jnp.float32),
            pltpu.VMEM((ROWS, 128), jnp.int32),
            pltpu.VMEM((128,), jnp.float32),
            pltpu.VMEM((SLICE,), jnp.float32),
            pltpu.VMEM((SLICE,), jnp.float32),
            pltpu.VMEM_SHARED((TBL,), jnp.float32),
            pltpu.VMEM_SHARED((TBL,), jnp.float32),
        ],
    )(_segment_kernel)
    sums2, counts2 = seg(dist.reshape(NW, ROWS, 128),
                         keys.reshape(NW, ROWS, 128))

    out = pl.pallas_call(
        _loss_kernel,
        out_shape=jax.ShapeDtypeStruct((1, 1), jnp.float32),
    )(sums2.reshape(128, 128), counts2.reshape(128, 128))
    return out.reshape(())
